# Initial kernel scaffold; baseline (speedup 1.0000x reference)
import jax
import jax.numpy as jnp
from jax.experimental import pallas as pl


def kernel(x, edge_index, batch, W1, b1, W2, b2, W3, b3, W4, b4, lin_W, lin_b):
    raise NotImplementedError("write your pallas kernel here")


# speedup vs baseline: 8.0065x; 8.0065x over previous
import functools

import jax
import jax.numpy as jnp
from jax import lax
from jax.experimental import pallas as pl
from jax.experimental.pallas import tpu as pltpu
from jax.experimental.pallas import tpu_sc as plsc

N = 10000
E = 320000
D = 128
G = 64
NCLS = 10

N_PAD = 10240
TRASH = N
NC, NS, L = 2, 16, 16
NW = NC * NS
CH = 128
NCHUNK = 79
EPW = CH * NCHUNK
E_PAD = EPW * NW
RPT = N_PAD // NS
BN = 1024
GRID = N_PAD // BN

_mesh = plsc.VectorSubcoreMesh(
    core_axis_name="c", subcore_axis_name="s", num_cores=NC, num_subcores=NS)


@functools.partial(
    pl.kernel,
    out_type=jax.ShapeDtypeStruct((NC * N_PAD, D), jnp.float32),
    mesh=_mesh,
    scratch_types=[
        pltpu.VMEM((CH,), jnp.int32),
        pltpu.VMEM((CH, D), jnp.float32),
        pltpu.VMEM_SHARED((N_PAD, D), jnp.float32),
    ],
)
def _sc_degree(dst_hbm, out_hbm, dst_v, buf_v, acc_sh):
    cid = lax.axis_index("c")
    sid = lax.axis_index("s")
    wid = sid * NC + cid

    def _fill(val):
        def body(i, _):
            for j in range(D // L):
                buf_v[i, pl.ds(j * L, L)] = jnp.full((L,), val, jnp.float32)
            return _
        lax.fori_loop(0, CH, body, 0)

    _fill(0.0)
    for k in range(RPT // CH):
        pltpu.sync_copy(buf_v, acc_sh.at[pl.ds(sid * RPT + k * CH, CH)])
    plsc.subcore_barrier()
    _fill(1.0)

    def chunk(j, _):
        base = wid * EPW + j * CH
        pltpu.sync_copy(dst_hbm.at[pl.ds(base, CH)], dst_v)
        pltpu.sync_copy(buf_v, acc_sh.at[dst_v], add=True)
        return _
    lax.fori_loop(0, NCHUNK, chunk, 0)

    plsc.subcore_barrier()
    for k in range(RPT // CH):
        r = sid * RPT + k * CH
        pltpu.sync_copy(acc_sh.at[pl.ds(r, CH)], buf_v)
        pltpu.sync_copy(buf_v, out_hbm.at[pl.ds(cid * N_PAD + r, CH)])


@functools.partial(
    pl.kernel,
    out_type=jax.ShapeDtypeStruct((NC * N_PAD, D), jnp.float32),
    mesh=_mesh,
    scratch_types=[
        pltpu.VMEM((CH,), jnp.int32),
        pltpu.VMEM((CH,), jnp.int32),
        pltpu.VMEM((CH, D), jnp.float32),
        pltpu.VMEM_SHARED((N_PAD, D), jnp.float32),
        pltpu.SemaphoreType.DMA,
    ],
)
def _sc_segsum(g_hbm, src_hbm, dst_hbm, out_hbm, src_v, dst_v, rows_v, acc_sh,
               sem):
    cid = lax.axis_index("c")
    sid = lax.axis_index("s")
    wid = sid * NC + cid

    def zero(i, _):
        for j in range(D // L):
            rows_v[i, pl.ds(j * L, L)] = jnp.zeros((L,), jnp.float32)
        return _
    lax.fori_loop(0, CH, zero, 0)
    for k in range(RPT // CH):
        pltpu.sync_copy(rows_v, acc_sh.at[pl.ds(sid * RPT + k * CH, CH)])
    plsc.subcore_barrier()

    def chunk(j, _):
        base = wid * EPW + j * CH
        pltpu.sync_copy(src_hbm.at[pl.ds(base, CH)], src_v)
        pltpu.sync_copy(dst_hbm.at[pl.ds(base, CH)], dst_v)
        pltpu.async_copy(g_hbm.at[src_v], rows_v, sem).wait()
        pltpu.sync_copy(rows_v, acc_sh.at[dst_v], add=True)
        return _
    lax.fori_loop(0, NCHUNK, chunk, 0)

    plsc.subcore_barrier()
    for k in range(RPT // CH):
        r = sid * RPT + k * CH
        pltpu.sync_copy(acc_sh.at[pl.ds(r, CH)], rows_v)
        pltpu.sync_copy(rows_v, out_hbm.at[pl.ds(cid * N_PAD + r, CH)])


def _tc_first(x_ref, w_ref, d0_ref, d1_ref, g_ref, dis_ref):
    deg = d0_ref[:, :1] + d1_ref[:, :1] + 1.0
    dis = lax.rsqrt(deg)
    dis_ref[...] = dis
    g_ref[...] = dis * jnp.dot(x_ref[...], w_ref[...],
                               preferred_element_type=jnp.float32)


def _tc_layer(a0_ref, a1_ref, g_ref, dis_ref, b_ref, w_ref, h_ref, gn_ref):
    dis = dis_ref[...]
    h = jnp.maximum(
        dis * (a0_ref[...] + a1_ref[...] + g_ref[...]) + b_ref[...], 0.0)
    h_ref[...] = h
    gn_ref[...] = dis * jnp.dot(h, w_ref[...],
                                preferred_element_type=jnp.float32)


def _tc_pool(a0_ref, a1_ref, g_ref, dis_ref, b_ref, x1_ref, x2_ref, bt_ref,
             lw_ref, lb_ref, out_ref, sums, cnts):
    i = pl.program_id(0)
    x3 = jnp.maximum(
        dis_ref[...] * (a0_ref[...] + a1_ref[...] + g_ref[...]) + b_ref[...],
        0.0)
    xs = jnp.concatenate([x1_ref[...], x2_ref[...], x3], axis=1)
    oh = (lax.broadcasted_iota(jnp.int32, (G, BN), 0)
          == bt_ref[...]).astype(jnp.float32)
    s_c = jnp.dot(oh, xs, preferred_element_type=jnp.float32)
    c_c = jnp.sum(oh, axis=1, keepdims=True)

    @pl.when(i == 0)
    def _():
        sums[...] = s_c
        cnts[...] = c_c

    @pl.when(i > 0)
    def _():
        sums[...] += s_c
        cnts[...] += c_c

    @pl.when(i == GRID - 1)
    def _():
        pooled = sums[...] / jnp.maximum(cnts[...], 1.0)
        logits = jnp.dot(pooled, lw_ref[...],
                         preferred_element_type=jnp.float32) + lb_ref[...]
        m = jnp.max(logits, axis=1, keepdims=True)
        lse = jnp.log(jnp.sum(jnp.exp(logits - m), axis=1, keepdims=True)) + m
        out_ref[...] = logits - lse


def _row_spec(off):
    return pl.BlockSpec((BN, D), lambda i, o=off: (i + o, 0))


def _full_spec(shape):
    return pl.BlockSpec(shape, lambda i: (0, 0))


def kernel(x, edge_index, batch, W1, b1, W2, b2, W3, b3, W4, b4, lin_W, lin_b):
    f32 = jnp.float32
    x_pad = jnp.pad(x, ((0, N_PAD - N), (0, 0)))
    src_pad = jnp.concatenate(
        [edge_index[0], jnp.zeros((E_PAD - E,), jnp.int32)])
    dst_pad = jnp.concatenate(
        [edge_index[1], jnp.full((E_PAD - E,), TRASH, jnp.int32)])
    bt2 = jnp.pad(batch, (0, N_PAD - N), constant_values=G)[None, :]
    b1r, b2r, b3r, b4r = (b[None, :] for b in (b1, b2, b3, b4))
    lbr = lin_b[None, :]

    degp = _sc_degree(dst_pad)

    g0, dis = pl.pallas_call(
        _tc_first,
        grid=(GRID,),
        in_specs=[
            _row_spec(0),
            _full_spec((D, D)),
            pl.BlockSpec((BN, D), lambda i: (i, 0)),
            pl.BlockSpec((BN, D), lambda i: (i + GRID, 0)),
        ],
        out_specs=[_row_spec(0), pl.BlockSpec((BN, 1), lambda i: (i, 0))],
        out_shape=[jax.ShapeDtypeStruct((N_PAD, D), f32),
                   jax.ShapeDtypeStruct((N_PAD, 1), f32)],
    )(x_pad, W1, degp, degp)

    def layer_call(acc, g, b, w):
        return pl.pallas_call(
            _tc_layer,
            grid=(GRID,),
            in_specs=[
                _row_spec(0), _row_spec(GRID), _row_spec(0),
                pl.BlockSpec((BN, 1), lambda i: (i, 0)),
                _full_spec((1, D)), _full_spec((D, D)),
            ],
            out_specs=[_row_spec(0), _row_spec(0)],
            out_shape=[jax.ShapeDtypeStruct((N_PAD, D), f32),
                       jax.ShapeDtypeStruct((N_PAD, D), f32)],
        )(acc, acc, g, dis, b, w)

    acc0 = _sc_segsum(g0, src_pad, dst_pad)
    x1, g1 = layer_call(acc0, g0, b1r, W2)
    acc1 = _sc_segsum(g1, src_pad, dst_pad)
    x2, g2 = layer_call(acc1, g1, b2r, W3)
    acc2 = _sc_segsum(g2, src_pad, dst_pad)
    _, g3 = layer_call(acc2, g2, b3r, W4)
    acc3 = _sc_segsum(g3, src_pad, dst_pad)

    out = pl.pallas_call(
        _tc_pool,
        grid=(GRID,),
        in_specs=[
            _row_spec(0), _row_spec(GRID), _row_spec(0),
            pl.BlockSpec((BN, 1), lambda i: (i, 0)),
            _full_spec((1, D)),
            _row_spec(0), _row_spec(0),
            pl.BlockSpec((1, BN), lambda i: (0, i)),
            _full_spec((3 * D, NCLS)), _full_spec((1, NCLS)),
        ],
        out_specs=pl.BlockSpec((G, NCLS), lambda i: (0, 0)),
        out_shape=jax.ShapeDtypeStruct((G, NCLS), f32),
        scratch_shapes=[pltpu.VMEM((G, 3 * D), f32), pltpu.VMEM((G, 1), f32)],
    )(acc3, acc3, g3, dis, b4r, x1, x2, bt2, lin_W, lbr)

    return out
